# Initial kernel scaffold; baseline (speedup 1.0000x reference)
import functools

import jax
import jax.numpy as jnp
from jax import lax
from jax.experimental import pallas as pl
from jax.experimental.pallas import tpu as pltpu
from jax.experimental.pallas import tpu_sc as plsc

N = 10000
E = 320000
D_IN = 128
D_HID = 128
D_OUT = 64

NC = 2
NS = 16
NW = NC * NS
EPT = E // NW
CHUNK = 80
NCHUNKS = EPT // CHUNK
N_PAD = 10240
ROWS_PT = N_PAD // NS
WB = 128
NWB = ROWS_PT // WB

_mesh = plsc.VectorSubcoreMesh(core_axis_name="c", subcore_axis_name="s")


@functools.partial(
    pl.kernel,
    out_type=[
        jax.ShapeDtypeStruct((NC, N_PAD, 16), jnp.float32),
        jax.ShapeDtypeStruct((NC, N_PAD, 16), jnp.float32),
    ],
    mesh=_mesh,
    scratch_types=[
        pltpu.VMEM((CHUNK,), jnp.int32),
        pltpu.VMEM((CHUNK,), jnp.int32),
        pltpu.VMEM((CHUNK, 16), jnp.float32),
        pltpu.VMEM((WB, 16), jnp.float32),
        pltpu.VMEM_SHARED((N_PAD, 16), jnp.float32),
        pltpu.VMEM_SHARED((N_PAD, 16), jnp.float32),
    ],
)
def _deg_kernel(src_hbm, dst_hbm, zeros_hbm, ones_hbm,
                dego_hbm, degi_hbm,
                sidx, didx, ones_v, wb, acc_o, acc_i):
    c = lax.axis_index("c")
    s = lax.axis_index("s")
    wid = c * NS + s
    r0 = s * ROWS_PT
    pltpu.sync_copy(zeros_hbm, wb)
    for k in range(NWB):
        pltpu.sync_copy(wb, acc_o.at[pl.ds(r0 + k * WB, WB)])
        pltpu.sync_copy(wb, acc_i.at[pl.ds(r0 + k * WB, WB)])
    pltpu.sync_copy(ones_hbm, ones_v)
    plsc.subcore_barrier()
    ebase = wid * EPT

    def body(j, carry):
        b = ebase + j * CHUNK
        pltpu.sync_copy(src_hbm.at[pl.ds(b, CHUNK)], sidx)
        pltpu.sync_copy(dst_hbm.at[pl.ds(b, CHUNK)], didx)
        pltpu.sync_copy(ones_v, acc_o.at[sidx], add=True)
        pltpu.sync_copy(ones_v, acc_i.at[didx], add=True)
        return carry

    lax.fori_loop(0, NCHUNKS, body, 0)
    plsc.subcore_barrier()
    for k in range(NWB):
        rr = r0 + k * WB
        pltpu.sync_copy(acc_o.at[pl.ds(rr, WB)], wb)
        pltpu.sync_copy(wb, dego_hbm.at[c, pl.ds(rr, WB)])
        pltpu.sync_copy(acc_i.at[pl.ds(rr, WB)], wb)
        pltpu.sync_copy(wb, degi_hbm.at[c, pl.ds(rr, WB)])


def _make_prop(D):
    @functools.partial(
        pl.kernel,
        out_type=jax.ShapeDtypeStruct((NC, N_PAD, D), jnp.float32),
        mesh=_mesh,
        scratch_types=[
            pltpu.VMEM((CHUNK,), jnp.int32),
            pltpu.VMEM((CHUNK,), jnp.int32),
            pltpu.VMEM((CHUNK, D), jnp.float32),
            pltpu.VMEM((WB, D), jnp.float32),
            pltpu.VMEM_SHARED((N_PAD, D), jnp.float32),
            pltpu.SemaphoreType.DMA,
        ],
    )
    def _prop(m_hbm, src_hbm, dst_hbm, zeros_hbm, out_hbm,
              sidx, didx, rows, wb, acc, sem):
        c = lax.axis_index("c")
        s = lax.axis_index("s")
        wid = c * NS + s
        r0 = s * ROWS_PT
        pltpu.sync_copy(zeros_hbm, wb)
        for k in range(NWB):
            pltpu.sync_copy(wb, acc.at[pl.ds(r0 + k * WB, WB)])
        plsc.subcore_barrier()
        ebase = wid * EPT

        def body(j, carry):
            b = ebase + j * CHUNK
            pltpu.sync_copy(src_hbm.at[pl.ds(b, CHUNK)], sidx)
            pltpu.sync_copy(dst_hbm.at[pl.ds(b, CHUNK)], didx)
            pltpu.async_copy(m_hbm.at[sidx], rows, sem).wait()
            pltpu.sync_copy(rows, acc.at[didx], add=True)
            return carry

        lax.fori_loop(0, NCHUNKS, body, 0)
        plsc.subcore_barrier()
        for k in range(NWB):
            rr = r0 + k * WB
            pltpu.sync_copy(acc.at[pl.ds(rr, WB)], wb)
            pltpu.sync_copy(wb, out_hbm.at[c, pl.ds(rr, WB)])

    return _prop


_prop128 = _make_prop(128)
_prop64 = _make_prop(64)

_BLK = 2000
_GRID = N // _BLK


def _norm_col(deg_blk):
    d = deg_blk[0] + deg_blk[1]
    return lax.rsqrt(jnp.maximum(d, 1.0))[:, 0:1]


def _tc1_body(x_ref, w_ref, dego_ref, out_ref):
    nsrc = _norm_col(dego_ref[...])
    out_ref[...] = jnp.dot(x_ref[...], w_ref[...],
                           preferred_element_type=jnp.float32) * nsrc


def _tc1(x, W1, dego):
    return pl.pallas_call(
        _tc1_body,
        grid=(_GRID,),
        in_specs=[
            pl.BlockSpec((_BLK, D_IN), lambda i: (i, 0)),
            pl.BlockSpec((D_IN, D_HID), lambda i: (0, 0)),
            pl.BlockSpec((NC, _BLK, 16), lambda i: (0, i, 0)),
        ],
        out_specs=pl.BlockSpec((_BLK, D_HID), lambda i: (i, 0)),
        out_shape=jax.ShapeDtypeStruct((N, D_HID), jnp.float32),
    )(x, W1, dego)


def _tc2_body(p_ref, dego_ref, degi_ref, w_ref, b_ref, out_ref):
    nsrc = _norm_col(dego_ref[...])
    ndst = _norm_col(degi_ref[...])
    agg = (p_ref[0] + p_ref[1]) * ndst
    h = jnp.maximum(agg + b_ref[...], 0.0)
    out_ref[...] = jnp.dot(h, w_ref[...],
                           preferred_element_type=jnp.float32) * nsrc


def _tc2(p1, dego, degi, W2, b1):
    return pl.pallas_call(
        _tc2_body,
        grid=(_GRID,),
        in_specs=[
            pl.BlockSpec((NC, _BLK, D_HID), lambda i: (0, i, 0)),
            pl.BlockSpec((NC, _BLK, 16), lambda i: (0, i, 0)),
            pl.BlockSpec((NC, _BLK, 16), lambda i: (0, i, 0)),
            pl.BlockSpec((D_HID, D_OUT), lambda i: (0, 0)),
            pl.BlockSpec((1, D_HID), lambda i: (0, 0)),
        ],
        out_specs=pl.BlockSpec((_BLK, D_OUT), lambda i: (i, 0)),
        out_shape=jax.ShapeDtypeStruct((N, D_OUT), jnp.float32),
    )(p1, dego, degi, W2, b1)


def _tc3_body(p_ref, degi_ref, b_ref, out_ref):
    ndst = _norm_col(degi_ref[...])
    out_ref[...] = (p_ref[0] + p_ref[1]) * ndst + b_ref[...]


def _tc3(p2, degi, b2):
    return pl.pallas_call(
        _tc3_body,
        grid=(_GRID,),
        in_specs=[
            pl.BlockSpec((NC, _BLK, D_OUT), lambda i: (0, i, 0)),
            pl.BlockSpec((NC, _BLK, 16), lambda i: (0, i, 0)),
            pl.BlockSpec((1, D_OUT), lambda i: (0, 0)),
        ],
        out_specs=pl.BlockSpec((_BLK, D_OUT), lambda i: (i, 0)),
        out_shape=jax.ShapeDtypeStruct((N, D_OUT), jnp.float32),
    )(p2, degi, b2)


def kernel(features, edge_index, W1, b1, W2, b2):
    src = edge_index[0]
    dst = edge_index[1]
    zeros16 = jnp.zeros((WB, 16), jnp.float32)
    ones16 = jnp.ones((CHUNK, 16), jnp.float32)
    zeros128 = jnp.zeros((WB, D_HID), jnp.float32)
    zeros64 = jnp.zeros((WB, D_OUT), jnp.float32)

    dego, degi = _deg_kernel(src, dst, zeros16, ones16)
    m1 = _tc1(features, W1, dego)
    p1 = _prop128(m1, src, dst, zeros128)
    m2 = _tc2(p1, dego, degi, W2, b1.reshape(1, D_HID))
    p2 = _prop64(m2, src, dst, zeros64)
    return _tc3(p2, degi, b2.reshape(1, D_OUT))


# speedup vs baseline: 5.0293x; 5.0293x over previous
import functools

import jax
import jax.numpy as jnp
from jax import lax
from jax.experimental import pallas as pl
from jax.experimental.pallas import tpu as pltpu
from jax.experimental.pallas import tpu_sc as plsc

N = 10000
E = 320000
D_IN = 128
D_HID = 128
D_OUT = 64

NC = 2
NS = 16
NW = NC * NS
EPT = E // NW
CHUNK = 80
NCHUNKS = EPT // CHUNK
N_PAD = 10240
ROWS_PT = N_PAD // NS
WB = 128
NWB = ROWS_PT // WB

_mesh = plsc.VectorSubcoreMesh(core_axis_name="c", subcore_axis_name="s")


@functools.partial(
    pl.kernel,
    out_type=[
        jax.ShapeDtypeStruct((NC, N_PAD, 16), jnp.float32),
        jax.ShapeDtypeStruct((NC, N_PAD, 16), jnp.float32),
    ],
    mesh=_mesh,
    compiler_params=pltpu.CompilerParams(use_tc_tiling_on_sc=False),
    scratch_types=[
        pltpu.VMEM((CHUNK,), jnp.int32),
        pltpu.VMEM((CHUNK,), jnp.int32),
        pltpu.VMEM((CHUNK, 16), jnp.float32),
        pltpu.VMEM((WB, 16), jnp.float32),
        pltpu.VMEM_SHARED((N_PAD, 16), jnp.float32),
        pltpu.VMEM_SHARED((N_PAD, 16), jnp.float32),
    ],
)
def _deg_kernel(src_hbm, dst_hbm, zeros_hbm, ones_hbm,
                dego_hbm, degi_hbm,
                sidx, didx, ones_v, wb, acc_o, acc_i):
    c = lax.axis_index("c")
    s = lax.axis_index("s")
    wid = c * NS + s
    r0 = s * ROWS_PT
    pltpu.sync_copy(zeros_hbm, wb)
    for k in range(NWB):
        pltpu.sync_copy(wb, acc_o.at[pl.ds(r0 + k * WB, WB)])
        pltpu.sync_copy(wb, acc_i.at[pl.ds(r0 + k * WB, WB)])
    pltpu.sync_copy(ones_hbm, ones_v)
    plsc.subcore_barrier()
    ebase = wid * EPT

    def body(j, carry):
        b = ebase + j * CHUNK
        pltpu.sync_copy(src_hbm.at[pl.ds(b, CHUNK)], sidx)
        pltpu.sync_copy(dst_hbm.at[pl.ds(b, CHUNK)], didx)
        pltpu.sync_copy(ones_v, acc_o.at[sidx], add=True)
        pltpu.sync_copy(ones_v, acc_i.at[didx], add=True)
        return carry

    lax.fori_loop(0, NCHUNKS, body, 0)
    plsc.subcore_barrier()
    for k in range(NWB):
        rr = r0 + k * WB
        pltpu.sync_copy(acc_o.at[pl.ds(rr, WB)], wb)
        pltpu.sync_copy(wb, dego_hbm.at[c, pl.ds(rr, WB)])
        pltpu.sync_copy(acc_i.at[pl.ds(rr, WB)], wb)
        pltpu.sync_copy(wb, degi_hbm.at[c, pl.ds(rr, WB)])


def _make_prop(D):
    @functools.partial(
        pl.kernel,
        out_type=jax.ShapeDtypeStruct((NC, N_PAD, D), jnp.float32),
        mesh=_mesh,
        compiler_params=pltpu.CompilerParams(use_tc_tiling_on_sc=False),
        scratch_types=[
            pltpu.VMEM((CHUNK,), jnp.int32),
            pltpu.VMEM((CHUNK,), jnp.int32),
            pltpu.VMEM((CHUNK, D), jnp.float32),
            pltpu.VMEM((WB, D), jnp.float32),
            pltpu.VMEM_SHARED((N_PAD, D), jnp.float32),
            pltpu.SemaphoreType.DMA,
        ],
    )
    def _prop(m_hbm, src_hbm, dst_hbm, zeros_hbm, out_hbm,
              sidx, didx, rows, wb, acc, sem):
        c = lax.axis_index("c")
        s = lax.axis_index("s")
        wid = c * NS + s
        r0 = s * ROWS_PT
        pltpu.sync_copy(zeros_hbm, wb)
        for k in range(NWB):
            pltpu.sync_copy(wb, acc.at[pl.ds(r0 + k * WB, WB)])
        plsc.subcore_barrier()
        ebase = wid * EPT

        def body(j, carry):
            b = ebase + j * CHUNK
            pltpu.sync_copy(src_hbm.at[pl.ds(b, CHUNK)], sidx)
            pltpu.sync_copy(dst_hbm.at[pl.ds(b, CHUNK)], didx)
            pltpu.async_copy(m_hbm.at[sidx], rows, sem).wait()
            pltpu.sync_copy(rows, acc.at[didx], add=True)
            return carry

        lax.fori_loop(0, NCHUNKS, body, 0)
        plsc.subcore_barrier()
        for k in range(NWB):
            rr = r0 + k * WB
            pltpu.sync_copy(acc.at[pl.ds(rr, WB)], wb)
            pltpu.sync_copy(wb, out_hbm.at[c, pl.ds(rr, WB)])

    return _prop


_prop128 = _make_prop(128)
_prop64 = _make_prop(64)

_BLK = 2000
_GRID = N // _BLK


def _norm_col(deg_blk):
    d = deg_blk[0] + deg_blk[1]
    return lax.rsqrt(jnp.maximum(d, 1.0))[:, 0:1]


def _tc1_body(x_ref, w_ref, dego_ref, out_ref):
    nsrc = _norm_col(dego_ref[...])
    out_ref[...] = jnp.dot(x_ref[...], w_ref[...],
                           preferred_element_type=jnp.float32) * nsrc


def _tc1(x, W1, dego):
    return pl.pallas_call(
        _tc1_body,
        grid=(_GRID,),
        in_specs=[
            pl.BlockSpec((_BLK, D_IN), lambda i: (i, 0)),
            pl.BlockSpec((D_IN, D_HID), lambda i: (0, 0)),
            pl.BlockSpec((NC, _BLK, 16), lambda i: (0, i, 0)),
        ],
        out_specs=pl.BlockSpec((_BLK, D_HID), lambda i: (i, 0)),
        out_shape=jax.ShapeDtypeStruct((N, D_HID), jnp.float32),
    )(x, W1, dego)


def _tc2_body(p_ref, dego_ref, degi_ref, w_ref, b_ref, out_ref):
    nsrc = _norm_col(dego_ref[...])
    ndst = _norm_col(degi_ref[...])
    agg = (p_ref[0] + p_ref[1]) * ndst
    h = jnp.maximum(agg + b_ref[...], 0.0)
    out_ref[...] = jnp.dot(h, w_ref[...],
                           preferred_element_type=jnp.float32) * nsrc


def _tc2(p1, dego, degi, W2, b1):
    return pl.pallas_call(
        _tc2_body,
        grid=(_GRID,),
        in_specs=[
            pl.BlockSpec((NC, _BLK, D_HID), lambda i: (0, i, 0)),
            pl.BlockSpec((NC, _BLK, 16), lambda i: (0, i, 0)),
            pl.BlockSpec((NC, _BLK, 16), lambda i: (0, i, 0)),
            pl.BlockSpec((D_HID, D_OUT), lambda i: (0, 0)),
            pl.BlockSpec((1, D_HID), lambda i: (0, 0)),
        ],
        out_specs=pl.BlockSpec((_BLK, D_OUT), lambda i: (i, 0)),
        out_shape=jax.ShapeDtypeStruct((N, D_OUT), jnp.float32),
    )(p1, dego, degi, W2, b1)


def _tc3_body(p_ref, degi_ref, b_ref, out_ref):
    ndst = _norm_col(degi_ref[...])
    out_ref[...] = (p_ref[0] + p_ref[1]) * ndst + b_ref[...]


def _tc3(p2, degi, b2):
    return pl.pallas_call(
        _tc3_body,
        grid=(_GRID,),
        in_specs=[
            pl.BlockSpec((NC, _BLK, D_OUT), lambda i: (0, i, 0)),
            pl.BlockSpec((NC, _BLK, 16), lambda i: (0, i, 0)),
            pl.BlockSpec((1, D_OUT), lambda i: (0, 0)),
        ],
        out_specs=pl.BlockSpec((_BLK, D_OUT), lambda i: (i, 0)),
        out_shape=jax.ShapeDtypeStruct((N, D_OUT), jnp.float32),
    )(p2, degi, b2)


def kernel(features, edge_index, W1, b1, W2, b2):
    src = edge_index[0]
    dst = edge_index[1]
    zeros16 = jnp.zeros((WB, 16), jnp.float32)
    ones16 = jnp.ones((CHUNK, 16), jnp.float32)
    zeros128 = jnp.zeros((WB, D_HID), jnp.float32)
    zeros64 = jnp.zeros((WB, D_OUT), jnp.float32)

    dego, degi = _deg_kernel(src, dst, zeros16, ones16)
    m1 = _tc1(features, W1, dego)
    p1 = _prop128(m1, src, dst, zeros128)
    m2 = _tc2(p1, dego, degi, W2, b1.reshape(1, D_HID))
    p2 = _prop64(m2, src, dst, zeros64)
    return _tc3(p2, degi, b2.reshape(1, D_OUT))
